# baseline (speedup 1.0000x reference)
import jax
import jax.numpy as jnp
from jax.experimental import pallas as pl
from jax.experimental.pallas import tpu as pltpu

D = 512
NN = 4096
CH = 1024
NC = NN // CH
CHUNKS = ((0, 512), (512, 512), (1024, 1024), (2048, 1024), (3072, 1024))
NCH = len(CHUNKS)


def _body(cell_ref, idx_ref, cs_ref, bm_ref, bl_ref, bu_ref, b1_ref, b2_ref,
          Wg_ref, bg_ref, ns_hbm, Wm_hbm, Wl_hbm, Wu_hbm, W1_hbm, W2_hbm,
          out_ref,
          ns_v, Wm_v, Wmb_v, Wl_v, Wu_v, W1_v, W2_v, mt_v,
          ns_sem, wm_sem, wl_sem, wu_sem, w1_sem, w2_sem):
    ns_cp = [pltpu.make_async_copy(ns_hbm.at[pl.ds(off, sz), :],
                                   ns_v.at[pl.ds(off, sz), :],
                                   ns_sem.at[i])
             for i, (off, sz) in enumerate(CHUNKS)]
    wm_cp = pltpu.make_async_copy(Wm_hbm, Wm_v, wm_sem)
    w_cp = [pltpu.make_async_copy(h, v, s) for h, v, s in
            ((Wl_hbm, Wl_v, wl_sem), (Wu_hbm, Wu_v, wu_sem),
             (W1_hbm, W1_v, w1_sem), (W2_hbm, W2_v, w2_sem))]

    ns_cp[0].start()
    wm_cp.start()
    ns_cp[1].start()
    ns_cp[2].start()

    cell = cell_ref[0]
    cx = (cell // 729).astype(jnp.float32)
    cy = ((cell // 27) % 27).astype(jnp.float32)
    cz = (cell % 27).astype(jnp.float32)

    local_sum = jnp.zeros((1, D), jnp.float32)
    dist_sum = jnp.zeros((1, D), jnp.float32)
    all_sum = jnp.zeros((1, D), jnp.float32)
    func_sum = jnp.zeros((1, D), jnp.float32)
    lc = 0.0
    fc = 0.0
    dc = 0.0

    row = jax.lax.broadcasted_iota(jnp.int32, (8, CH), 0)
    for c in range(NC):
        idx = idx_ref[c].astype(jnp.float32)
        nx = jnp.floor(idx * (1.0 / 729.0))
        r = idx - 729.0 * nx
        ny = jnp.floor(r * (1.0 / 27.0))
        nz = r - 27.0 * ny
        d2 = (nx - cx) ** 2 + (ny - cy) ** 2 + (nz - cz) ** 2
        local_m = jnp.where(d2 <= 3.24, 1.0, 0.0)
        dist_m = jnp.where(d2 > 36.0, 1.0, 0.0)
        func_m = 1.0 - local_m - dist_m

        lc += jnp.sum(local_m)
        fc += jnp.sum(func_m)
        dc += jnp.sum(dist_m)

        M = jnp.where(row == 0, local_m, jnp.where(row == 1, func_m,
            jnp.where(row == 2, dist_m, 0.0)))
        mt_v[pl.ds(c * CH, CH), :] = jnp.transpose(M, (1, 0))

    wm_cp.wait()
    Wmb_v[...] = Wm_v[...].astype(jnp.bfloat16)

    for c, (off, sz) in enumerate(CHUNKS):
        Mt = mt_v[pl.ds(off, sz), :]
        lm_col = Mt[:, 0:1]
        fm_col = Mt[:, 1:2]
        dm_col = Mt[:, 2:3]

        ns_cp[c].wait()
        if c + 3 < NCH:
            ns_cp[c + 3].start()
        if c == 0:
            w_cp[0].start()
            w_cp[1].start()
        elif c == 1:
            w_cp[2].start()
            w_cp[3].start()
        ns = ns_v[pl.ds(off, sz), :]

        local_sum += jnp.sum(ns * lm_col, axis=0, keepdims=True)
        dist_sum += jnp.sum(ns * dm_col, axis=0, keepdims=True)
        all_sum += jnp.sum(ns, axis=0, keepdims=True)

        t = jnp.tanh(jax.lax.dot_general(
            ns.astype(jnp.bfloat16), Wmb_v[...], (((1,), (0,)), ((), ())),
            preferred_element_type=jnp.float32) + bm_ref[...])
        func_sum += jnp.sum(t * fm_col, axis=0, keepdims=True)

    local_agg = local_sum / jnp.maximum(lc, 1.0)
    func_agg = func_sum / jnp.maximum(fc, 1.0)
    dist_agg = dist_sum / jnp.maximum(dc, 1.0)
    all_agg = all_sum * (1.0 / NN)

    cs = cs_ref[...]

    def mm(a, w):
        return jax.lax.dot_general(a, w, (((1,), (0,)), ((), ())),
                                   preferred_element_type=jnp.float32)

    w_cp[0].wait()
    xl = jnp.concatenate([cs, local_agg], axis=1)
    local_out = jnp.tanh(mm(xl, Wl_v[...]) + bl_ref[...])

    w_cp[1].wait()
    xf = jnp.concatenate([cs, func_agg], axis=1)
    func_out = jnp.tanh(mm(xf, Wu_v[...]) + bu_ref[...])

    w_cp[2].wait()
    w_cp[3].wait()
    z = cs
    for _ in range(3):
        h = jnp.tanh(mm(jnp.concatenate([z, dist_agg], axis=1), W1_v[...])
                     + b1_ref[...])
        z = z + 0.3 * (mm(h, W2_v[...]) + b2_ref[...])

    logits = mm(jnp.concatenate([cs, all_agg], axis=1), Wg_ref[...]) + bg_ref[...]
    m = jnp.max(logits, axis=1, keepdims=True)
    e = jnp.exp(logits - m)
    g = e / jnp.sum(e, axis=1, keepdims=True)

    out_ref[...] = (g[:, 0:1] * local_out + g[:, 1:2] * func_out
                    + g[:, 2:3] * z)


def kernel(current_state, neighbor_states, cell_idx, neighbor_indices,
           W_local, b_local, W_msg, b_msg, W_upd, b_upd,
           W_cnf1, b_cnf1, W_cnf2, b_cnf2, W_gate, b_gate):
    cell = jnp.reshape(jnp.asarray(cell_idx, dtype=jnp.int32), (1,))
    idx3 = jnp.reshape(neighbor_indices.astype(jnp.int32), (NC, 1, CH))
    cs = jnp.reshape(current_state, (1, D))

    full = lambda shape: pl.BlockSpec(shape, lambda: (0,) * len(shape))
    any_spec = pl.BlockSpec(memory_space=pl.ANY)
    out = pl.pallas_call(
        _body,
        in_specs=[
            pl.BlockSpec(memory_space=pltpu.SMEM),
            full((NC, 1, CH)),
            full((1, D)),
            full((1, D)),
            full((1, D)),
            full((1, D)),
            full((1, 2 * D)),
            full((1, D)),
            full((2 * D, 3)),
            full((1, 3)),
            any_spec,
            any_spec,
            any_spec,
            any_spec,
            any_spec,
            any_spec,
        ],
        out_specs=pl.BlockSpec((1, D), lambda: (0, 0)),
        out_shape=jax.ShapeDtypeStruct((1, D), jnp.float32),
        scratch_shapes=[
            pltpu.VMEM((NN, D), jnp.float32),
            pltpu.VMEM((D, D), jnp.float32),
            pltpu.VMEM((D, D), jnp.bfloat16),
            pltpu.VMEM((2 * D, D), jnp.float32),
            pltpu.VMEM((2 * D, D), jnp.float32),
            pltpu.VMEM((2 * D, 2 * D), jnp.float32),
            pltpu.VMEM((2 * D, D), jnp.float32),
            pltpu.VMEM((NN, 8), jnp.float32),
            pltpu.SemaphoreType.DMA((NCH,)),
            pltpu.SemaphoreType.DMA,
            pltpu.SemaphoreType.DMA,
            pltpu.SemaphoreType.DMA,
            pltpu.SemaphoreType.DMA,
            pltpu.SemaphoreType.DMA,
        ],
    )(cell, idx3, cs, jnp.reshape(b_msg, (1, D)), jnp.reshape(b_local, (1, D)),
      jnp.reshape(b_upd, (1, D)), jnp.reshape(b_cnf1, (1, 2 * D)),
      jnp.reshape(b_cnf2, (1, D)), W_gate, jnp.reshape(b_gate, (1, 3)),
      neighbor_states, W_msg, W_local, W_upd, W_cnf1, W_cnf2)
    return jnp.reshape(out, (D,))


# speedup vs baseline: 1.0114x; 1.0114x over previous
import jax
import jax.numpy as jnp
from jax.experimental import pallas as pl
from jax.experimental.pallas import tpu as pltpu

D = 512
NN = 4096
CH = 1024
NC = NN // CH


def _body(cell_ref, idx_ref, cs_ref, bm_ref, bl_ref, bu_ref, b1_ref, b2_ref,
          Wg_ref, bg_ref, ns_hbm, Wm_hbm, Wl_hbm, Wu_hbm, W1_hbm, W2_hbm,
          out_ref,
          ns_v, Wm_v, Wmb_v, Wl_v, Wu_v, W1_v, W2_v, mt_v,
          ns_sem, wm_sem, wl_sem, wu_sem, w1_sem, w2_sem):
    ns_cp = [pltpu.make_async_copy(ns_hbm.at[pl.ds(c * CH, CH), :],
                                   ns_v.at[pl.ds(c * CH, CH), :],
                                   ns_sem.at[c]) for c in range(NC)]
    wm_cp = pltpu.make_async_copy(Wm_hbm, Wm_v, wm_sem)
    w_cp = [pltpu.make_async_copy(h, v, s) for h, v, s in
            ((Wl_hbm, Wl_v, wl_sem), (Wu_hbm, Wu_v, wu_sem),
             (W1_hbm, W1_v, w1_sem), (W2_hbm, W2_v, w2_sem))]

    ns_cp[0].start()
    wm_cp.start()
    ns_cp[1].start()
    ns_cp[2].start()

    cell = cell_ref[0]
    cx = (cell // 729).astype(jnp.float32)
    cy = ((cell // 27) % 27).astype(jnp.float32)
    cz = (cell % 27).astype(jnp.float32)

    local_sum = jnp.zeros((1, D), jnp.float32)
    dist_sum = jnp.zeros((1, D), jnp.float32)
    all_sum = jnp.zeros((1, D), jnp.float32)
    func_sum = jnp.zeros((1, D), jnp.float32)
    lc = 0.0
    fc = 0.0
    dc = 0.0

    row = jax.lax.broadcasted_iota(jnp.int32, (8, CH), 0)
    for c in range(NC):
        idx = idx_ref[c].astype(jnp.float32)
        nx = jnp.floor(idx * (1.0 / 729.0))
        r = idx - 729.0 * nx
        ny = jnp.floor(r * (1.0 / 27.0))
        nz = r - 27.0 * ny
        d2 = (nx - cx) ** 2 + (ny - cy) ** 2 + (nz - cz) ** 2
        local_m = jnp.where(d2 <= 3.24, 1.0, 0.0)
        dist_m = jnp.where(d2 > 36.0, 1.0, 0.0)
        func_m = 1.0 - local_m - dist_m

        lc += jnp.sum(local_m)
        fc += jnp.sum(func_m)
        dc += jnp.sum(dist_m)

        M = jnp.where(row == 0, local_m, jnp.where(row == 1, func_m,
            jnp.where(row == 2, dist_m, 0.0)))
        mt_v[pl.ds(c * CH, CH), :] = jnp.transpose(M, (1, 0))

    wm_cp.wait()
    Wmb_v[...] = Wm_v[...].astype(jnp.bfloat16)

    for c in range(NC):
        Mt = mt_v[pl.ds(c * CH, CH), :]
        lm_col = Mt[:, 0:1]
        fm_col = Mt[:, 1:2]
        dm_col = Mt[:, 2:3]

        ns_cp[c].wait()
        if c + 3 < NC:
            ns_cp[c + 3].start()
        if c == 0:
            w_cp[0].start()
            w_cp[1].start()
        elif c == 1:
            w_cp[2].start()
            w_cp[3].start()
        ns = ns_v[pl.ds(c * CH, CH), :]

        local_sum += jnp.sum(ns * lm_col, axis=0, keepdims=True)
        dist_sum += jnp.sum(ns * dm_col, axis=0, keepdims=True)
        all_sum += jnp.sum(ns, axis=0, keepdims=True)

        t = jnp.tanh(jax.lax.dot_general(
            ns.astype(jnp.bfloat16), Wmb_v[...], (((1,), (0,)), ((), ())),
            preferred_element_type=jnp.float32) + bm_ref[...])
        func_sum += jnp.sum(t * fm_col, axis=0, keepdims=True)

    local_agg = local_sum / jnp.maximum(lc, 1.0)
    func_agg = func_sum / jnp.maximum(fc, 1.0)
    dist_agg = dist_sum / jnp.maximum(dc, 1.0)
    all_agg = all_sum * (1.0 / NN)

    cs = cs_ref[...]

    def mm(a, w):
        return jax.lax.dot_general(a, w, (((1,), (0,)), ((), ())),
                                   preferred_element_type=jnp.float32)

    w_cp[0].wait()
    xl = jnp.concatenate([cs, local_agg], axis=1)
    local_out = jnp.tanh(mm(xl, Wl_v[...]) + bl_ref[...])

    w_cp[1].wait()
    xf = jnp.concatenate([cs, func_agg], axis=1)
    func_out = jnp.tanh(mm(xf, Wu_v[...]) + bu_ref[...])

    w_cp[2].wait()
    w_cp[3].wait()
    z = cs
    for _ in range(3):
        h = jnp.tanh(mm(jnp.concatenate([z, dist_agg], axis=1), W1_v[...])
                     + b1_ref[...])
        z = z + 0.3 * (mm(h, W2_v[...]) + b2_ref[...])

    logits = mm(jnp.concatenate([cs, all_agg], axis=1), Wg_ref[...]) + bg_ref[...]
    m = jnp.max(logits, axis=1, keepdims=True)
    e = jnp.exp(logits - m)
    g = e / jnp.sum(e, axis=1, keepdims=True)

    out_ref[...] = (g[:, 0:1] * local_out + g[:, 1:2] * func_out
                    + g[:, 2:3] * z)


def kernel(current_state, neighbor_states, cell_idx, neighbor_indices,
           W_local, b_local, W_msg, b_msg, W_upd, b_upd,
           W_cnf1, b_cnf1, W_cnf2, b_cnf2, W_gate, b_gate):
    cell = jnp.reshape(jnp.asarray(cell_idx, dtype=jnp.int32), (1,))
    idx3 = jnp.reshape(neighbor_indices.astype(jnp.int32), (NC, 1, CH))
    cs = jnp.reshape(current_state, (1, D))

    full = lambda shape: pl.BlockSpec(shape, lambda: (0,) * len(shape))
    any_spec = pl.BlockSpec(memory_space=pl.ANY)
    out = pl.pallas_call(
        _body,
        in_specs=[
            pl.BlockSpec(memory_space=pltpu.SMEM),
            full((NC, 1, CH)),
            full((1, D)),
            full((1, D)),
            full((1, D)),
            full((1, D)),
            full((1, 2 * D)),
            full((1, D)),
            full((2 * D, 3)),
            full((1, 3)),
            any_spec,
            any_spec,
            any_spec,
            any_spec,
            any_spec,
            any_spec,
        ],
        out_specs=pl.BlockSpec((1, D), lambda: (0, 0)),
        out_shape=jax.ShapeDtypeStruct((1, D), jnp.float32),
        scratch_shapes=[
            pltpu.VMEM((NN, D), jnp.float32),
            pltpu.VMEM((D, D), jnp.float32),
            pltpu.VMEM((D, D), jnp.bfloat16),
            pltpu.VMEM((2 * D, D), jnp.float32),
            pltpu.VMEM((2 * D, D), jnp.float32),
            pltpu.VMEM((2 * D, 2 * D), jnp.float32),
            pltpu.VMEM((2 * D, D), jnp.float32),
            pltpu.VMEM((NN, 8), jnp.float32),
            pltpu.SemaphoreType.DMA((NC,)),
            pltpu.SemaphoreType.DMA,
            pltpu.SemaphoreType.DMA,
            pltpu.SemaphoreType.DMA,
            pltpu.SemaphoreType.DMA,
            pltpu.SemaphoreType.DMA,
        ],
    )(cell, idx3, cs, jnp.reshape(b_msg, (1, D)), jnp.reshape(b_local, (1, D)),
      jnp.reshape(b_upd, (1, D)), jnp.reshape(b_cnf1, (1, 2 * D)),
      jnp.reshape(b_cnf2, (1, D)), W_gate, jnp.reshape(b_gate, (1, 3)),
      neighbor_states, W_msg, W_local, W_upd, W_cnf1, W_cnf2)
    return jnp.reshape(out, (D,))
